# baseline (speedup 1.0000x reference)
import functools

import jax
import jax.numpy as jnp
from jax import lax
from jax.experimental import pallas as pl
from jax.experimental.pallas import tpu as pltpu
from jax.experimental.pallas import tpu_sc as plsc

_NW = 32
_LANES = 16


def _pooled_lookup(items2, table, B, S, D):
    bpw = B // _NW
    chunk = S // 2
    nlg = D // _LANES
    mesh = plsc.VectorSubcoreMesh(core_axis_name="c", subcore_axis_name="s")

    @functools.partial(
        pl.kernel,
        out_type=jax.ShapeDtypeStruct((B, D), jnp.float32),
        mesh=mesh,
        scratch_types=[
            pltpu.VMEM((2 * bpw, chunk), jnp.int32),
            pltpu.VMEM((chunk, D), jnp.float32),
            pltpu.VMEM((chunk, D), jnp.float32),
            pltpu.VMEM((bpw, D), jnp.float32),
            pltpu.SemaphoreType.DMA,
            pltpu.SemaphoreType.DMA,
        ],
        compiler_params=pltpu.CompilerParams(use_tc_tiling_on_sc=False),
    )
    def k(table_hbm, items_hbm, out_hbm, idx_v, buf0, buf1, out_v, sem0, sem1):
        wid = lax.axis_index("s") * 2 + lax.axis_index("c")
        base = wid * bpw
        pltpu.sync_copy(items_hbm.at[pl.ds(2 * base, 2 * bpw)], idx_v)

        inv = jnp.float32(1.0 / S)

        @pl.loop(0, bpw)
        def _(b):
            c0 = pltpu.async_copy(table_hbm.at[idx_v.at[2 * b]], buf0, sem0)
            c1 = pltpu.async_copy(table_hbm.at[idx_v.at[2 * b + 1]], buf1, sem1)
            c0.wait()
            c1.wait()

            def body(r, accs):
                return tuple(
                    accs[g]
                    + buf0[r, pl.ds(_LANES * g, _LANES)]
                    + buf1[r, pl.ds(_LANES * g, _LANES)]
                    for g in range(nlg)
                )

            accs = tuple(jnp.zeros((_LANES,), jnp.float32) for _ in range(nlg))
            accs = lax.fori_loop(0, chunk, body, accs)
            for g in range(nlg):
                out_v[b, pl.ds(_LANES * g, _LANES)] = accs[g] * inv

        pltpu.sync_copy(out_v, out_hbm.at[pl.ds(base, bpw)])

    return k(table, items2)


def kernel(items, table):
    B, S = items.shape
    _, D = table.shape
    items2 = items.reshape(B * 2, S // 2).astype(jnp.int32)
    return _pooled_lookup(items2, table, B, S, D)


# speedup vs baseline: 1.0906x; 1.0906x over previous
import functools

import jax
import jax.numpy as jnp
from jax import lax
from jax.experimental import pallas as pl
from jax.experimental.pallas import tpu as pltpu
from jax.experimental.pallas import tpu_sc as plsc

_NW = 32
_LANES = 16


def _pooled_lookup(items, table, B, S, D):
    bpw = B // _NW
    nlg = D // _LANES
    mesh = plsc.VectorSubcoreMesh(core_axis_name="c", subcore_axis_name="s")

    @functools.partial(
        pl.kernel,
        out_type=jax.ShapeDtypeStruct((B, D), jnp.float32),
        mesh=mesh,
        scratch_types=[
            pltpu.VMEM((bpw, S), jnp.int32),
            pltpu.VMEM((S, D), jnp.float32),
            pltpu.VMEM((S, D), jnp.float32),
            pltpu.VMEM((bpw, D), jnp.float32),
            pltpu.SemaphoreType.DMA,
            pltpu.SemaphoreType.DMA,
        ],
        compiler_params=pltpu.CompilerParams(use_tc_tiling_on_sc=False),
    )
    def k(table_hbm, items_hbm, out_hbm, idx_v, buf0, buf1, out_v, sem0, sem1):
        wid = lax.axis_index("s") * 2 + lax.axis_index("c")
        base = wid * bpw
        pltpu.sync_copy(items_hbm.at[pl.ds(base, bpw)], idx_v)

        inv = jnp.float32(1.0 / S)

        def accumulate(buf, b):
            def body(r, accs):
                return tuple(
                    accs[g] + buf[r, pl.ds(_LANES * g, _LANES)] for g in range(nlg)
                )

            accs = tuple(jnp.zeros((_LANES,), jnp.float32) for _ in range(nlg))
            accs = lax.fori_loop(0, S, body, accs)
            for g in range(nlg):
                out_v[b, pl.ds(_LANES * g, _LANES)] = accs[g] * inv

        pltpu.async_copy(table_hbm.at[idx_v.at[0]], buf0, sem0)

        @pl.loop(0, bpw, step=2)
        def _(b):
            pltpu.async_copy(table_hbm.at[idx_v.at[b + 1]], buf1, sem1)
            pltpu.make_async_copy(table_hbm.at[idx_v.at[b]], buf0, sem0).wait()
            accumulate(buf0, b)

            @pl.when(b + 2 < bpw)
            def _():
                pltpu.async_copy(table_hbm.at[idx_v.at[b + 2]], buf0, sem0)

            pltpu.make_async_copy(table_hbm.at[idx_v.at[b + 1]], buf1, sem1).wait()
            accumulate(buf1, b + 1)

        pltpu.sync_copy(out_v, out_hbm.at[pl.ds(base, bpw)])

    return k(table, items)


def kernel(items, table):
    B, S = items.shape
    _, D = table.shape
    return _pooled_lookup(items.astype(jnp.int32), table, B, S, D)
